# baseline (speedup 1.0000x reference)
import jax
import jax.numpy as jnp
from jax import lax
from jax.experimental import pallas as pl
from jax.experimental.pallas import tpu as pltpu
from jax.experimental.pallas import tpu_sc as plsc

D_MODEL = 768
SEQ = 8192
BATCH = 4

NC = 2
NS = 16
NW = NC * NS

ROWS_PER_W = SEQ // NW
CH = 8
N_CHUNKS = ROWS_PER_W // CH
CHW = CH * D_MODEL
G = 8
N_GROUPS = CHW // (16 * G)


def _body(x_hbm, pos_hbm, out_hbm,
          xb0, xb1, ob0, ob1, pb0, pb1,
          sx0, sx1, so0, so1, sp0, sp1):
    xb = (xb0, xb1)
    ob = (ob0, ob1)
    pb = (pb0, pb1)
    sx = (sx0, sx1)
    so = (so0, so1)
    sp = (sp0, sp1)

    w = lax.axis_index("s") * NC + lax.axis_index("c")
    base = w * ROWS_PER_W

    def start_in(c, slot):
        r0 = base + c * CH
        pltpu.async_copy(pos_hbm.at[pl.ds(r0, CH), :], pb[slot], sp[slot])
        for b in range(BATCH):
            pltpu.async_copy(x_hbm.at[b, pl.ds(r0, CH), :],
                             xb[slot].at[b], sx[slot])

    def wait_in(c, slot):
        r0 = base + c * CH
        pltpu.make_async_copy(pos_hbm.at[pl.ds(r0, CH), :],
                              pb[slot], sp[slot]).wait()
        for b in range(BATCH):
            pltpu.make_async_copy(x_hbm.at[b, pl.ds(r0, CH), :],
                                  xb[slot].at[b], sx[slot]).wait()

    def start_out(c, slot):
        r0 = base + c * CH
        for b in range(BATCH):
            pltpu.async_copy(ob[slot].at[b],
                             out_hbm.at[b, pl.ds(r0, CH), :], so[slot])

    def wait_out(c, slot):
        r0 = base + c * CH
        for b in range(BATCH):
            pltpu.make_async_copy(ob[slot].at[b],
                                  out_hbm.at[b, pl.ds(r0, CH), :],
                                  so[slot]).wait()

    def compute(slot):
        xs, os_, ps = xb[slot], ob[slot], pb[slot]

        @plsc.parallel_loop(0, CH)
        def _(row):
            for h in range(D_MODEL // (16 * G)):
                c0 = h * (16 * G)
                pos_vecs = [ps[row, pl.ds(c0 + k * 16, 16)] for k in range(G)]
                for b in range(BATCH):
                    for k in range(G):
                        sl = pl.ds(c0 + k * 16, 16)
                        os_[b, row, sl] = xs[b, row, sl] + pos_vecs[k]

    start_in(0, 0)
    start_in(1, 1)
    for c in (0, 1):
        wait_in(c, c)
        compute(c)
        start_out(c, c)
        start_in(c + 2, c)

    @pl.loop(2, N_CHUNKS, step=2)
    def _(c0):
        for d in range(2):
            c = c0 + d
            wait_in(c, d)
            wait_out(c - 2, d)
            compute(d)
            start_out(c, d)

            @pl.when(c + 2 < N_CHUNKS)
            def _():
                start_in(c + 2, d)

    wait_out(N_CHUNKS - 2, 0)
    wait_out(N_CHUNKS - 1, 1)


@jax.jit
def kernel(x, pos_embedding):
    seq = x.shape[1]
    pos = pos_embedding[:seq]
    mesh = plsc.VectorSubcoreMesh(core_axis_name="c", subcore_axis_name="s")
    return pl.kernel(
        _body,
        mesh=mesh,
        out_type=jax.ShapeDtypeStruct(x.shape, jnp.float32),
        scratch_types=[
            pltpu.VMEM((BATCH, CH, D_MODEL), jnp.float32),
            pltpu.VMEM((BATCH, CH, D_MODEL), jnp.float32),
            pltpu.VMEM((BATCH, CH, D_MODEL), jnp.float32),
            pltpu.VMEM((BATCH, CH, D_MODEL), jnp.float32),
            pltpu.VMEM((CH, D_MODEL), jnp.float32),
            pltpu.VMEM((CH, D_MODEL), jnp.float32),
            pltpu.SemaphoreType.DMA,
            pltpu.SemaphoreType.DMA,
            pltpu.SemaphoreType.DMA,
            pltpu.SemaphoreType.DMA,
            pltpu.SemaphoreType.DMA,
            pltpu.SemaphoreType.DMA,
        ],
    )(x, pos)


# speedup vs baseline: 1.4525x; 1.4525x over previous
import jax
import jax.numpy as jnp
from jax import lax
from jax.experimental import pallas as pl
from jax.experimental.pallas import tpu as pltpu
from jax.experimental.pallas import tpu_sc as plsc

D_MODEL = 768
SEQ = 8192
BATCH = 4

NC = 2
NS = 16
NW = NC * NS

ROWS_PER_W = SEQ // NW
CH = 8
N_CHUNKS = ROWS_PER_W // CH
CHW = CH * D_MODEL
G = 8
N_GROUPS = CHW // (16 * G)


def _body(x_hbm, pos_hbm, out_hbm,
          xb0, xb1, ob0, ob1, pb0, pb1,
          sx0, sx1, so0, so1, sp0, sp1):
    xb = (xb0, xb1)
    ob = (ob0, ob1)
    pb = (pb0, pb1)
    sx = (sx0, sx1)
    so = (so0, so1)
    sp = (sp0, sp1)

    w = lax.axis_index("s") * NC + lax.axis_index("c")
    base = w * ROWS_PER_W

    def in_copies(c, slot):
        r0 = base + c * CH
        copies = [pltpu.make_async_copy(
            pos_hbm.at[r0 + r, :],
            pb[slot].at[pl.ds(r * D_MODEL, D_MODEL)], sp[slot])
            for r in range(CH)]
        copies += [pltpu.make_async_copy(
            x_hbm.at[b, r0 + r, :],
            xb[slot].at[pl.ds(b * CHW + r * D_MODEL, D_MODEL)], sx[slot])
            for b in range(BATCH) for r in range(CH)]
        return copies

    def out_copies(c, slot):
        r0 = base + c * CH
        return [pltpu.make_async_copy(
            ob[slot].at[pl.ds(b * CHW + r * D_MODEL, D_MODEL)],
            out_hbm.at[b, r0 + r, :], so[slot])
            for b in range(BATCH) for r in range(CH)]

    def start_in(c, slot):
        for cp in in_copies(c, slot):
            cp.start()

    def wait_in(c, slot):
        for cp in in_copies(c, slot):
            cp.wait()

    def start_out(c, slot):
        for cp in out_copies(c, slot):
            cp.start()

    def wait_out(c, slot):
        for cp in out_copies(c, slot):
            cp.wait()

    def compute(slot):
        xs, os_, ps = xb[slot], ob[slot], pb[slot]

        @plsc.parallel_loop(0, N_GROUPS)
        def _(i):
            gbase = i * (16 * G)
            pos_vecs = [ps[pl.ds(gbase + k * 16, 16)] for k in range(G)]
            for b in range(BATCH):
                for k in range(G):
                    sl = pl.ds(b * CHW + gbase + k * 16, 16)
                    os_[sl] = xs[sl] + pos_vecs[k]

    start_in(0, 0)
    start_in(1, 1)
    for c in (0, 1):
        wait_in(c, c)
        compute(c)
        start_out(c, c)
        start_in(c + 2, c)

    @pl.loop(2, N_CHUNKS, step=2)
    def _(c0):
        for d in range(2):
            c = c0 + d
            wait_in(c, d)
            wait_out(c - 2, d)
            compute(d)
            start_out(c, d)

            @pl.when(c + 2 < N_CHUNKS)
            def _():
                start_in(c + 2, d)

    wait_out(N_CHUNKS - 2, 0)
    wait_out(N_CHUNKS - 1, 1)


@jax.jit
def kernel(x, pos_embedding):
    seq = x.shape[1]
    pos = pos_embedding[:seq]
    mesh = plsc.VectorSubcoreMesh(core_axis_name="c", subcore_axis_name="s")
    return pl.kernel(
        _body,
        mesh=mesh,
        out_type=jax.ShapeDtypeStruct(x.shape, jnp.float32),
        scratch_types=[
            pltpu.VMEM((BATCH * CHW,), jnp.float32),
            pltpu.VMEM((BATCH * CHW,), jnp.float32),
            pltpu.VMEM((BATCH * CHW,), jnp.float32),
            pltpu.VMEM((BATCH * CHW,), jnp.float32),
            pltpu.VMEM((CHW,), jnp.float32),
            pltpu.VMEM((CHW,), jnp.float32),
            pltpu.SemaphoreType.DMA,
            pltpu.SemaphoreType.DMA,
            pltpu.SemaphoreType.DMA,
            pltpu.SemaphoreType.DMA,
            pltpu.SemaphoreType.DMA,
            pltpu.SemaphoreType.DMA,
        ],
    )(x, pos)


# speedup vs baseline: 1.4855x; 1.0227x over previous
import jax
import jax.numpy as jnp
from jax import lax
from jax.experimental import pallas as pl
from jax.experimental.pallas import tpu as pltpu
from jax.experimental.pallas import tpu_sc as plsc

D_MODEL = 768
SEQ = 8192
BATCH = 4

NC = 2
NS = 16
NW = NC * NS

ROWS_PER_W = SEQ // NW
CH = 8
N_CHUNKS = ROWS_PER_W // CH
CHW = CH * D_MODEL
G = 8
N_GROUPS = CHW // (16 * G)


def _body(x_hbm, pos_hbm, dummy_hbm, out_hbm,
          xb0, xb1, ob0, ob1, pb0, pb1,
          sx0, sx1, so0, so1, sp0, sp1):
    xb = (xb0, xb1)
    ob = (ob0, ob1)
    pb = (pb0, pb1)
    sx = (sx0, sx1)
    so = (so0, so1)
    sp = (sp0, sp1)

    w = lax.axis_index("s") * NC + lax.axis_index("c")
    base = w * ROWS_PER_W

    def in_copies(c, slot):
        r0 = base + c * CH
        copies = [pltpu.make_async_copy(
            pos_hbm.at[r0 + r, :],
            pb[slot].at[pl.ds(r * D_MODEL, D_MODEL)], sp[slot])
            for r in range(CH)]
        copies += [pltpu.make_async_copy(
            x_hbm.at[b, r0 + r, :],
            xb[slot].at[pl.ds(b * CHW + r * D_MODEL, D_MODEL)], sx[slot])
            for b in range(BATCH) for r in range(CH)]
        return copies

    def out_copies(c, slot):
        r0 = base + c * CH
        return [pltpu.make_async_copy(
            ob[slot].at[pl.ds(b * CHW + r * D_MODEL, D_MODEL)],
            out_hbm.at[b, r0 + r, :], so[slot])
            for b in range(BATCH) for r in range(CH)]

    def start_in(c, slot):
        for cp in in_copies(c, slot):
            cp.start()

    def wait_in(c, slot):
        pltpu.make_async_copy(dummy_hbm.at[pl.ds(0, CHW)],
                              pb[slot], sp[slot]).wait()
        pltpu.make_async_copy(dummy_hbm, xb[slot], sx[slot]).wait()

    def start_out(c, slot):
        for cp in out_copies(c, slot):
            cp.start()

    def wait_out(c, slot):
        pltpu.make_async_copy(ob[slot], dummy_hbm, so[slot]).wait()

    def compute(slot):
        xs, os_, ps = xb[slot], ob[slot], pb[slot]

        @plsc.parallel_loop(0, N_GROUPS)
        def _(i):
            gbase = i * (16 * G)
            pos_vecs = [ps[pl.ds(gbase + k * 16, 16)] for k in range(G)]
            for b in range(BATCH):
                for k in range(G):
                    sl = pl.ds(b * CHW + gbase + k * 16, 16)
                    os_[sl] = xs[sl] + pos_vecs[k]

    start_in(0, 0)
    start_in(1, 1)
    for c in (0, 1):
        wait_in(c, c)
        compute(c)
        start_out(c, c)
        start_in(c + 2, c)

    @pl.loop(2, N_CHUNKS, step=2)
    def _(c0):
        for d in range(2):
            c = c0 + d
            wait_in(c, d)
            wait_out(c - 2, d)
            compute(d)
            start_out(c, d)

            @pl.when(c + 2 < N_CHUNKS)
            def _():
                start_in(c + 2, d)

    wait_out(N_CHUNKS - 2, 0)
    wait_out(N_CHUNKS - 1, 1)


@jax.jit
def kernel(x, pos_embedding):
    seq = x.shape[1]
    pos = pos_embedding[:seq]
    dummy = jnp.zeros((BATCH * CHW,), jnp.float32)
    mesh = plsc.VectorSubcoreMesh(core_axis_name="c", subcore_axis_name="s")
    return pl.kernel(
        _body,
        mesh=mesh,
        out_type=jax.ShapeDtypeStruct(x.shape, jnp.float32),
        scratch_types=[
            pltpu.VMEM((BATCH * CHW,), jnp.float32),
            pltpu.VMEM((BATCH * CHW,), jnp.float32),
            pltpu.VMEM((BATCH * CHW,), jnp.float32),
            pltpu.VMEM((BATCH * CHW,), jnp.float32),
            pltpu.VMEM((CHW,), jnp.float32),
            pltpu.VMEM((CHW,), jnp.float32),
            pltpu.SemaphoreType.DMA,
            pltpu.SemaphoreType.DMA,
            pltpu.SemaphoreType.DMA,
            pltpu.SemaphoreType.DMA,
            pltpu.SemaphoreType.DMA,
            pltpu.SemaphoreType.DMA,
        ],
    )(x, pos, dummy)
